# baseline (speedup 1.0000x reference)
import functools

import numpy as np

import jax
import jax.numpy as jnp
from jax import lax
from jax.experimental import pallas as pl
from jax.experimental.pallas import tpu as pltpu
from jax.experimental.pallas import tpu_sc as plsc

N = 10000
E = 320000
ND = 128
ED = 16
D = 128
L = 3

NC = 2
NS = 16
NW = NC * NS
EPW = E // NW
B = 40
ITERS = EPW // B
NP = 10240
RPT = NP // NS

_f32 = jnp.float32
_bf16 = jnp.bfloat16

_LO_IDX = np.concatenate([np.arange(32 * j, 32 * j + 16) for j in range(D // 32)])
_HI_IDX = np.concatenate([np.arange(32 * j + 16, 32 * j + 32) for j in range(D // 32)])


def _mm_bias_body(x_ref, w_ref, b_ref, o_ref):
    o_ref[...] = (
        jnp.dot(x_ref[...], w_ref[...], preferred_element_type=_f32) + b_ref[...]
    )


def _mm_bias(x, w, b2, block_rows):
    m, k = x.shape
    n = w.shape[1]
    return pl.pallas_call(
        _mm_bias_body,
        grid=(m // block_rows,),
        in_specs=[
            pl.BlockSpec((block_rows, k), lambda i: (i, 0)),
            pl.BlockSpec((k, n), lambda i: (0, 0)),
            pl.BlockSpec((1, n), lambda i: (0, 0)),
        ],
        out_specs=pl.BlockSpec((block_rows, n), lambda i: (i, 0)),
        out_shape=jax.ShapeDtypeStruct((m, n), _f32),
    )(x, w, b2)


def _pack_table(u, wlo_ref, whi_ref):
    lo = jnp.dot(u, wlo_ref[...], preferred_element_type=_f32).astype(_bf16)
    hi = jnp.dot(u, whi_ref[...], preferred_element_type=_f32).astype(_bf16)
    lo32 = jax.lax.bitcast_convert_type(lo, jnp.uint16).astype(jnp.uint32)
    hi32 = jax.lax.bitcast_convert_type(hi, jnp.uint16).astype(jnp.uint32)
    return lo32 | (hi32 << 16)


def _enc_body(nf_ref, w_ref, b_ref, wsl_ref, wsh_ref, wdl_ref, wdh_ref,
              unf_ref, ps_ref, pd_ref):
    u = jnp.dot(nf_ref[...], w_ref[...], preferred_element_type=_f32) + b_ref[...]
    unf_ref[...] = u
    ps_ref[...] = _pack_table(u, wsl_ref, wsh_ref)
    pd_ref[...] = _pack_table(u, wdl_ref, wdh_ref)


def _encode_nodes(nf, w, b2, wsl, wsh, wdl, wdh, block_rows=2000):
    n_out = jax.ShapeDtypeStruct((N, D), _f32)
    t_out = jax.ShapeDtypeStruct((N, D // 2), jnp.uint32)
    half = pl.BlockSpec((D, D // 2), lambda i: (0, 0))
    return pl.pallas_call(
        _enc_body,
        grid=(N // block_rows,),
        in_specs=[
            pl.BlockSpec((block_rows, ND), lambda i: (i, 0)),
            pl.BlockSpec((ND, D), lambda i: (0, 0)),
            pl.BlockSpec((1, D), lambda i: (0, 0)),
            half, half, half, half,
        ],
        out_specs=[
            pl.BlockSpec((block_rows, D), lambda i: (i, 0)),
            pl.BlockSpec((block_rows, D // 2), lambda i: (i, 0)),
            pl.BlockSpec((block_rows, D // 2), lambda i: (i, 0)),
        ],
        out_shape=[n_out, t_out, t_out],
    )(nf, w, b2, wsl, wsh, wdl, wdh)


def _fold_body(wenc_ref, wee_ref, benc_ref, be_ref, wf_ref, bf_ref):
    wf_ref[...] = jnp.dot(wenc_ref[...], wee_ref[...], preferred_element_type=_f32)
    bf_ref[...] = (
        jnp.dot(benc_ref[...], wee_ref[...], preferred_element_type=_f32) + be_ref[...]
    )


def _fold_edge_encoder(w_edge_enc, we_e0, benc8, be0):
    return pl.pallas_call(
        _fold_body,
        grid=(1,),
        in_specs=[
            pl.BlockSpec((ED, D), lambda i: (0, 0)),
            pl.BlockSpec((D, D), lambda i: (0, 0)),
            pl.BlockSpec((8, D), lambda i: (0, 0)),
            pl.BlockSpec((1, D), lambda i: (0, 0)),
        ],
        out_specs=[
            pl.BlockSpec((ED, D), lambda i: (0, 0)),
            pl.BlockSpec((8, D), lambda i: (0, 0)),
        ],
        out_shape=[
            jax.ShapeDtypeStruct((ED, D), _f32),
            jax.ShapeDtypeStruct((8, D), _f32),
        ],
    )(w_edge_enc, we_e0, benc8, be0)


def _node_body(unf_ref, a0_ref, a1_ref, wn_ref, wa_ref, b_ref,
               wsl_ref, wsh_ref, wdl_ref, wdh_ref,
               out_ref, ps_ref, pd_ref):
    agg = a0_ref[...] + a1_ref[...]
    u = (
        jnp.dot(unf_ref[...], wn_ref[...], preferred_element_type=_f32)
        + jnp.dot(agg, wa_ref[...], preferred_element_type=_f32)
        + b_ref[...]
    )
    u = jnp.maximum(u, 0.0)
    out_ref[...] = u
    ps_ref[...] = _pack_table(u, wsl_ref, wsh_ref)
    pd_ref[...] = _pack_table(u, wdl_ref, wdh_ref)


def _node_last_body(unf_ref, a0_ref, a1_ref, wn_ref, wa_ref, b_ref, out_ref):
    agg = a0_ref[...] + a1_ref[...]
    u = (
        jnp.dot(unf_ref[...], wn_ref[...], preferred_element_type=_f32)
        + jnp.dot(agg, wa_ref[...], preferred_element_type=_f32)
        + b_ref[...]
    )
    out_ref[...] = jnp.maximum(u, 0.0)


def _node_update(unf, a0, a1, wn, wa, b2, wsl, wsh, wdl, wdh, block_rows=2000):
    n_out = jax.ShapeDtypeStruct((N, D), _f32)
    t_out = jax.ShapeDtypeStruct((N, D // 2), jnp.uint32)
    row = pl.BlockSpec((block_rows, D), lambda i: (i, 0))
    trow = pl.BlockSpec((block_rows, D // 2), lambda i: (i, 0))
    full = pl.BlockSpec((D, D), lambda i: (0, 0))
    half = pl.BlockSpec((D, D // 2), lambda i: (0, 0))
    return pl.pallas_call(
        _node_body,
        grid=(N // block_rows,),
        in_specs=[row, row, row, full, pl.BlockSpec((D, D), lambda i: (0, 0)),
                  pl.BlockSpec((1, D), lambda i: (0, 0)), half, half, half, half],
        out_specs=[row, trow, trow],
        out_shape=[n_out, t_out, t_out],
    )(unf, a0, a1, wn, wa, b2, wsl, wsh, wdl, wdh)


def _node_update_last(unf, a0, a1, wn, wa, b2, block_rows=2000):
    row = pl.BlockSpec((block_rows, D), lambda i: (i, 0))
    full = pl.BlockSpec((D, D), lambda i: (0, 0))
    return pl.pallas_call(
        _node_last_body,
        grid=(N // block_rows,),
        in_specs=[row, row, row, full, pl.BlockSpec((D, D), lambda i: (0, 0)),
                  pl.BlockSpec((1, D), lambda i: (0, 0))],
        out_specs=row,
        out_shape=jax.ShapeDtypeStruct((N, D), _f32),
    )(unf, a0, a1, wn, wa, b2)


def _sc_edge_body(z_hbm, src_hbm, dst_hbm, ps_hbm, pd_hbm,
                  uef_hbm, aggp_hbm,
                  si0_v, si1_v, di0_v, di1_v,
                  z0_v, z1_v, ps0_v, ps1_v, pd0_v, pd1_v,
                  agg_sh,
                  sem_i0, sem_i1, sem_z0, sem_z1, sem_ps0, sem_ps1,
                  sem_pd0, sem_pd1, sem_w0, sem_w1):
    c = lax.axis_index("c")
    s = lax.axis_index("s")
    wid = s * NC + c

    slots = (
        (si0_v, di0_v, z0_v, ps0_v, pd0_v, sem_i0, sem_z0, sem_ps0, sem_pd0,
         sem_w0),
        (si1_v, di1_v, z1_v, ps1_v, pd1_v, sem_i1, sem_z1, sem_ps1, sem_pd1,
         sem_w1),
    )

    def _zrow(r, carry):
        for cc in range(D // 16):
            z0_v[r, pl.ds(cc * 16, 16)] = jnp.zeros((16,), _f32)
        return carry

    lax.fori_loop(0, B, _zrow, 0)
    for j in range(RPT // B):
        pltpu.sync_copy(z0_v, agg_sh.at[pl.ds(s * RPT + j * B, B)])
    plsc.subcore_barrier()

    def _issue_idx(k, slot):
        si_b, di_b = slots[slot][0], slots[slot][1]
        s_i = slots[slot][5]

        @pl.when(k < ITERS)
        def _():
            base = wid * EPW + k * B
            pltpu.async_copy(src_hbm.at[pl.ds(base, B)], si_b, s_i)
            pltpu.async_copy(dst_hbm.at[pl.ds(base, B)], di_b, s_i)

    def _issue_data(k, slot):
        si_b, di_b, z_b, ps_b, pd_b, s_i, s_z, s_ps, s_pd, s_w = slots[slot]

        @pl.when(k < ITERS)
        def _():
            base = wid * EPW + k * B
            pltpu.make_async_copy(src_hbm.at[pl.ds(base, B)], si_b, s_i).wait()
            pltpu.make_async_copy(dst_hbm.at[pl.ds(base, B)], di_b, s_i).wait()

            @pl.when(k >= 2)
            def _():
                pltpu.make_async_copy(z_b, uef_hbm.at[pl.ds(0, B)], s_w).wait()

            pltpu.async_copy(z_hbm.at[pl.ds(base, B)], z_b, s_z)
            pltpu.async_copy(ps_hbm.at[si_b], ps_b, s_ps)
            pltpu.async_copy(pd_hbm.at[di_b], pd_b, s_pd)

    def _stage(k, slot, other):
        si_b, di_b, z_b, ps_b, pd_b, s_i, s_z, s_ps, s_pd, s_w = slots[slot]
        _issue_data(k + 1, other)

        base = wid * EPW + k * B
        pltpu.make_async_copy(z_hbm.at[pl.ds(base, B)], z_b, s_z).wait()
        pltpu.make_async_copy(ps_hbm.at[si_b], ps_b, s_ps).wait()
        pltpu.make_async_copy(pd_hbm.at[di_b], pd_b, s_pd).wait()

        hi_mask = jnp.uint32(0xFFFF0000)

        def _row(r, inner):
            for cc in range(D // 32):
                pw = ps_b[r, pl.ds(cc * 16, 16)]
                qw = pd_b[r, pl.ds(cc * 16, 16)]
                pa = lax.bitcast_convert_type(pw << 16, _f32)
                pb = lax.bitcast_convert_type(pw & hi_mask, _f32)
                qa = lax.bitcast_convert_type(qw << 16, _f32)
                qb = lax.bitcast_convert_type(qw & hi_mask, _f32)
                sl0 = pl.ds(cc * 32, 16)
                sl1 = pl.ds(cc * 32 + 16, 16)
                z_b[r, sl0] = jnp.maximum(z_b[r, sl0] + pa + qa, 0.0)
                z_b[r, sl1] = jnp.maximum(z_b[r, sl1] + pb + qb, 0.0)
            return inner

        lax.fori_loop(0, B, _row, 0)
        pltpu.async_copy(z_b, uef_hbm.at[pl.ds(base, B)], s_w)
        pltpu.sync_copy(z_b, agg_sh.at[di_b], add=True)
        _issue_idx(k + 2, slot)

    _issue_idx(0, 0)
    _issue_idx(1, 1)
    _issue_data(0, 0)

    def _pair(j, carry):
        k0 = 2 * j
        _stage(k0, 0, 1)
        _stage(k0 + 1, 1, 0)
        return carry

    lax.fori_loop(0, ITERS // 2, _pair, 0)

    pltpu.make_async_copy(z0_v, uef_hbm.at[pl.ds(0, B)], sem_w0).wait()
    pltpu.make_async_copy(z1_v, uef_hbm.at[pl.ds(0, B)], sem_w1).wait()

    plsc.subcore_barrier()
    pltpu.sync_copy(agg_sh.at[pl.ds(s * RPT, RPT)],
                    aggp_hbm.at[pl.ds(c * NP + s * RPT, RPT)])


def _sc_edge(z, src, dst, ps_t, pd_t):
    mesh = plsc.VectorSubcoreMesh(
        core_axis_name="c", subcore_axis_name="s", num_cores=NC, num_subcores=NS
    )
    kern = pl.kernel(
        _sc_edge_body,
        compiler_params=pltpu.CompilerParams(use_tc_tiling_on_sc=False),
        out_type=(
            jax.ShapeDtypeStruct((E, D), _f32),
            jax.ShapeDtypeStruct((NC * NP, D), _f32),
        ),
        mesh=mesh,
        scratch_types=(
            [pltpu.VMEM((B,), jnp.int32)] * 4
            + [pltpu.VMEM((B, D), _f32)] * 2
            + [pltpu.VMEM((B, D // 2), jnp.uint32)] * 4
            + [pltpu.VMEM_SHARED((NP, D), _f32)]
            + [pltpu.SemaphoreType.DMA] * 10
        ),
    )
    return kern(z, src, dst, ps_t, pd_t)


def kernel(nf, ef, edge_index, W_node_enc, b_node_enc, W_edge_enc, b_edge_enc,
           We, be, Wn, bn):
    src = edge_index[0]
    dst = edge_index[1]

    ws = We[0, D:2 * D, :]
    wd = We[0, 2 * D:, :]
    unf, ps_t, pd_t = _encode_nodes(
        nf, W_node_enc, b_node_enc.reshape(1, D),
        ws[:, _LO_IDX], ws[:, _HI_IDX], wd[:, _LO_IDX], wd[:, _HI_IDX])

    benc8 = jnp.zeros((8, D), _f32).at[0].set(b_edge_enc)
    wf, bf8 = _fold_edge_encoder(W_edge_enc, We[0, :D, :], benc8,
                                 be[0].reshape(1, D))
    z = _mm_bias(ef, wf, bf8[0:1], block_rows=512)

    uef = None
    for l in range(L):
        if l > 0:
            z = _mm_bias(uef, We[l, :D, :], be[l].reshape(1, D), block_rows=512)
        uef, aggp = _sc_edge(z, src, dst, ps_t, pd_t)
        a0 = aggp[:N]
        a1 = aggp[NP:NP + N]
        if l < L - 1:
            ws = We[l + 1, D:2 * D, :]
            wd = We[l + 1, 2 * D:, :]
            unf, ps_t, pd_t = _node_update(
                unf, a0, a1, Wn[l, :D, :], Wn[l, D:, :], bn[l].reshape(1, D),
                ws[:, _LO_IDX], ws[:, _HI_IDX], wd[:, _LO_IDX], wd[:, _HI_IDX])
        else:
            unf = _node_update_last(
                unf, a0, a1, Wn[l, :D, :], Wn[l, D:, :], bn[l].reshape(1, D))
    return unf, uef


# speedup vs baseline: 1.0427x; 1.0427x over previous
import functools

import numpy as np

import jax
import jax.numpy as jnp
from jax import lax
from jax.experimental import pallas as pl
from jax.experimental.pallas import tpu as pltpu
from jax.experimental.pallas import tpu_sc as plsc

N = 10000
E = 320000
ND = 128
ED = 16
D = 128
L = 3

NC = 2
NS = 16
NW = NC * NS
EPW = E // NW
B = 40
ITERS = EPW // B
NP = 10240
RPT = NP // NS

_f32 = jnp.float32
_bf16 = jnp.bfloat16


def _mm_fuse_body(x_ref, w_ref, b_ref, g_ref, o_ref):
    o_ref[...] = jnp.maximum(
        jnp.dot(x_ref[...], w_ref[...], preferred_element_type=_f32)
        + g_ref[...] + b_ref[...],
        0.0,
    )


def _mm_fuse(x, w, b2, g, block_rows=512):
    m, k = x.shape
    n = w.shape[1]
    return pl.pallas_call(
        _mm_fuse_body,
        grid=(m // block_rows,),
        in_specs=[
            pl.BlockSpec((block_rows, k), lambda i: (i, 0)),
            pl.BlockSpec((k, n), lambda i: (0, 0)),
            pl.BlockSpec((1, n), lambda i: (0, 0)),
            pl.BlockSpec((block_rows, n), lambda i: (i, 0)),
        ],
        out_specs=pl.BlockSpec((block_rows, n), lambda i: (i, 0)),
        out_shape=jax.ShapeDtypeStruct((m, n), _f32),
    )(x, w, b2, g)


def _enc_body(nf_ref, w_ref, b_ref, ws_ref, wd_ref, unf_ref, ps_ref, pd_ref):
    u = jnp.dot(nf_ref[...], w_ref[...], preferred_element_type=_f32) + b_ref[...]
    unf_ref[...] = u
    ps_ref[...] = jnp.dot(u, ws_ref[...], preferred_element_type=_f32)
    pd_ref[...] = jnp.dot(u, wd_ref[...], preferred_element_type=_f32)


def _encode_nodes(nf, w, b2, ws, wd, block_rows=2000):
    n_out = jax.ShapeDtypeStruct((N, D), _f32)
    full = pl.BlockSpec((D, D), lambda i: (0, 0))
    return pl.pallas_call(
        _enc_body,
        grid=(N // block_rows,),
        in_specs=[
            pl.BlockSpec((block_rows, ND), lambda i: (i, 0)),
            pl.BlockSpec((ND, D), lambda i: (0, 0)),
            pl.BlockSpec((1, D), lambda i: (0, 0)),
            full, full,
        ],
        out_specs=[pl.BlockSpec((block_rows, D), lambda i: (i, 0))] * 3,
        out_shape=[n_out, n_out, n_out],
    )(nf, w, b2, ws, wd)


def _fold_body(wenc_ref, wee_ref, benc_ref, be_ref, wf_ref, bf_ref):
    wf_ref[...] = jnp.dot(wenc_ref[...], wee_ref[...], preferred_element_type=_f32)
    bf_ref[...] = (
        jnp.dot(benc_ref[...], wee_ref[...], preferred_element_type=_f32) + be_ref[...]
    )


def _fold_edge_encoder(w_edge_enc, we_e0, benc8, be0):
    return pl.pallas_call(
        _fold_body,
        grid=(1,),
        in_specs=[
            pl.BlockSpec((ED, D), lambda i: (0, 0)),
            pl.BlockSpec((D, D), lambda i: (0, 0)),
            pl.BlockSpec((8, D), lambda i: (0, 0)),
            pl.BlockSpec((1, D), lambda i: (0, 0)),
        ],
        out_specs=[
            pl.BlockSpec((ED, D), lambda i: (0, 0)),
            pl.BlockSpec((8, D), lambda i: (0, 0)),
        ],
        out_shape=[
            jax.ShapeDtypeStruct((ED, D), _f32),
            jax.ShapeDtypeStruct((8, D), _f32),
        ],
    )(w_edge_enc, we_e0, benc8, be0)


def _node_body(unf_ref, a0_ref, a1_ref, wn_ref, wa_ref, b_ref, ws_ref, wd_ref,
               out_ref, ps_ref, pd_ref):
    agg = a0_ref[...] + a1_ref[...]
    u = (
        jnp.dot(unf_ref[...], wn_ref[...], preferred_element_type=_f32)
        + jnp.dot(agg, wa_ref[...], preferred_element_type=_f32)
        + b_ref[...]
    )
    u = jnp.maximum(u, 0.0)
    out_ref[...] = u
    ps_ref[...] = jnp.dot(u, ws_ref[...], preferred_element_type=_f32)
    pd_ref[...] = jnp.dot(u, wd_ref[...], preferred_element_type=_f32)


def _node_last_body(unf_ref, a0_ref, a1_ref, wn_ref, wa_ref, b_ref, out_ref):
    agg = a0_ref[...] + a1_ref[...]
    u = (
        jnp.dot(unf_ref[...], wn_ref[...], preferred_element_type=_f32)
        + jnp.dot(agg, wa_ref[...], preferred_element_type=_f32)
        + b_ref[...]
    )
    out_ref[...] = jnp.maximum(u, 0.0)


def _node_update(unf, a0, a1, wn, wa, b2, ws, wd, block_rows=2000):
    n_out = jax.ShapeDtypeStruct((N, D), _f32)
    row = pl.BlockSpec((block_rows, D), lambda i: (i, 0))
    full = pl.BlockSpec((D, D), lambda i: (0, 0))
    return pl.pallas_call(
        _node_body,
        grid=(N // block_rows,),
        in_specs=[row, row, row, full, pl.BlockSpec((D, D), lambda i: (0, 0)),
                  pl.BlockSpec((1, D), lambda i: (0, 0)), full, full],
        out_specs=[row, row, row],
        out_shape=[n_out, n_out, n_out],
    )(unf, a0, a1, wn, wa, b2, ws, wd)


def _node_update_last(unf, a0, a1, wn, wa, b2, block_rows=2000):
    row = pl.BlockSpec((block_rows, D), lambda i: (i, 0))
    full = pl.BlockSpec((D, D), lambda i: (0, 0))
    return pl.pallas_call(
        _node_last_body,
        grid=(N // block_rows,),
        in_specs=[row, row, row, full, pl.BlockSpec((D, D), lambda i: (0, 0)),
                  pl.BlockSpec((1, D), lambda i: (0, 0))],
        out_specs=row,
        out_shape=jax.ShapeDtypeStruct((N, D), _f32),
    )(unf, a0, a1, wn, wa, b2)


def _mesh():
    return plsc.VectorSubcoreMesh(
        core_axis_name="c", subcore_axis_name="s", num_cores=NC, num_subcores=NS
    )


def _sc_gather_body(src_hbm, dst_hbm, ps_hbm, pd_hbm, g_hbm,
                    si0_v, si1_v, di0_v, di1_v,
                    ps0_v, ps1_v, pd0_v, pd1_v,
                    sem_i0, sem_i1, sem_ps0, sem_ps1, sem_pd0, sem_pd1,
                    sem_w0, sem_w1):
    c = lax.axis_index("c")
    s = lax.axis_index("s")
    wid = s * NC + c

    slots = (
        (si0_v, di0_v, ps0_v, pd0_v, sem_i0, sem_ps0, sem_pd0, sem_w0),
        (si1_v, di1_v, ps1_v, pd1_v, sem_i1, sem_ps1, sem_pd1, sem_w1),
    )

    def _issue_idx(k, slot):
        si_b, di_b = slots[slot][0], slots[slot][1]
        s_i = slots[slot][4]

        @pl.when(k < ITERS)
        def _():
            base = wid * EPW + k * B
            pltpu.async_copy(src_hbm.at[pl.ds(base, B)], si_b, s_i)
            pltpu.async_copy(dst_hbm.at[pl.ds(base, B)], di_b, s_i)

    def _issue_data(k, slot):
        si_b, di_b, ps_b, pd_b, s_i, s_ps, s_pd, s_w = slots[slot]

        @pl.when(k < ITERS)
        def _():
            base = wid * EPW + k * B
            pltpu.make_async_copy(src_hbm.at[pl.ds(base, B)], si_b, s_i).wait()
            pltpu.make_async_copy(dst_hbm.at[pl.ds(base, B)], di_b, s_i).wait()

            @pl.when(k >= 2)
            def _():
                pltpu.make_async_copy(ps_b, g_hbm.at[pl.ds(0, B)], s_w).wait()

            pltpu.async_copy(ps_hbm.at[si_b], ps_b, s_ps)
            pltpu.async_copy(pd_hbm.at[di_b], pd_b, s_pd)

    def _stage(k, slot, other):
        si_b, di_b, ps_b, pd_b, s_i, s_ps, s_pd, s_w = slots[slot]
        _issue_data(k + 1, other)

        base = wid * EPW + k * B
        pltpu.make_async_copy(ps_hbm.at[si_b], ps_b, s_ps).wait()
        pltpu.make_async_copy(pd_hbm.at[di_b], pd_b, s_pd).wait()

        def _row(r, inner):
            for cc in range(D // 16):
                sl = pl.ds(cc * 16, 16)
                ps_b[r, sl] = ps_b[r, sl] + pd_b[r, sl]
            return inner

        lax.fori_loop(0, B, _row, 0)
        pltpu.async_copy(ps_b, g_hbm.at[pl.ds(base, B)], s_w)
        _issue_idx(k + 2, slot)

    _issue_idx(0, 0)
    _issue_idx(1, 1)
    _issue_data(0, 0)

    def _pair(j, carry):
        k0 = 2 * j
        _stage(k0, 0, 1)
        _stage(k0 + 1, 1, 0)
        return carry

    lax.fori_loop(0, ITERS // 2, _pair, 0)
    pltpu.make_async_copy(ps0_v, g_hbm.at[pl.ds(0, B)], sem_w0).wait()
    pltpu.make_async_copy(ps1_v, g_hbm.at[pl.ds(0, B)], sem_w1).wait()


def _sc_gather(src, dst, ps_t, pd_t):
    kern = pl.kernel(
        _sc_gather_body,
        out_type=jax.ShapeDtypeStruct((E, D), _f32),
        mesh=_mesh(),
        scratch_types=(
            [pltpu.VMEM((B,), jnp.int32)] * 4
            + [pltpu.VMEM((B, D), _f32)] * 4
            + [pltpu.SemaphoreType.DMA] * 8
        ),
    )
    return kern(src, dst, ps_t, pd_t)


def _sc_scatter_body(uef_hbm, dst_hbm, aggp_hbm,
                     di0_v, di1_v, u0_v, u1_v, agg_sh,
                     sem_i0, sem_i1, sem_u0, sem_u1):
    c = lax.axis_index("c")
    s = lax.axis_index("s")
    wid = s * NC + c

    slots = (
        (di0_v, u0_v, sem_i0, sem_u0),
        (di1_v, u1_v, sem_i1, sem_u1),
    )

    def _zrow(r, carry):
        for cc in range(D // 16):
            u0_v[r, pl.ds(cc * 16, 16)] = jnp.zeros((16,), _f32)
        return carry

    lax.fori_loop(0, B, _zrow, 0)
    for j in range(RPT // B):
        pltpu.sync_copy(u0_v, agg_sh.at[pl.ds(s * RPT + j * B, B)])
    plsc.subcore_barrier()

    def _issue_idx(k, slot):
        di_b, _, s_i, _ = slots[slot]

        @pl.when(k < ITERS)
        def _():
            base = wid * EPW + k * B
            pltpu.async_copy(dst_hbm.at[pl.ds(base, B)], di_b, s_i)

    def _issue_data(k, slot):
        di_b, u_b, s_i, s_u = slots[slot]

        @pl.when(k < ITERS)
        def _():
            base = wid * EPW + k * B
            pltpu.make_async_copy(dst_hbm.at[pl.ds(base, B)], di_b, s_i).wait()
            pltpu.async_copy(uef_hbm.at[pl.ds(base, B)], u_b, s_u)

    def _stage(k, slot, other):
        di_b, u_b, s_i, s_u = slots[slot]
        _issue_data(k + 1, other)
        base = wid * EPW + k * B
        pltpu.make_async_copy(uef_hbm.at[pl.ds(base, B)], u_b, s_u).wait()
        pltpu.sync_copy(u_b, agg_sh.at[di_b], add=True)
        _issue_idx(k + 2, slot)

    _issue_idx(0, 0)
    _issue_idx(1, 1)
    _issue_data(0, 0)

    def _pair(j, carry):
        k0 = 2 * j
        _stage(k0, 0, 1)
        _stage(k0 + 1, 1, 0)
        return carry

    lax.fori_loop(0, ITERS // 2, _pair, 0)

    plsc.subcore_barrier()
    pltpu.sync_copy(agg_sh.at[pl.ds(s * RPT, RPT)],
                    aggp_hbm.at[pl.ds(c * NP + s * RPT, RPT)])


def _sc_scatter(uef, dst):
    kern = pl.kernel(
        _sc_scatter_body,
        out_type=jax.ShapeDtypeStruct((NC * NP, D), _f32),
        mesh=_mesh(),
        scratch_types=(
            [pltpu.VMEM((B,), jnp.int32)] * 2
            + [pltpu.VMEM((B, D), _f32)] * 2
            + [pltpu.VMEM_SHARED((NP, D), _f32)]
            + [pltpu.SemaphoreType.DMA] * 4
        ),
    )
    return kern(uef, dst)


def kernel(nf, ef, edge_index, W_node_enc, b_node_enc, W_edge_enc, b_edge_enc,
           We, be, Wn, bn):
    src = edge_index[0]
    dst = edge_index[1]

    unf, ps_t, pd_t = _encode_nodes(
        nf, W_node_enc, b_node_enc.reshape(1, D),
        We[0, D:2 * D, :], We[0, 2 * D:, :])

    benc8 = jnp.zeros((8, D), _f32).at[0].set(b_edge_enc)
    wf, bf8 = _fold_edge_encoder(W_edge_enc, We[0, :D, :], benc8,
                                 be[0].reshape(1, D))

    x, w, b2 = ef, wf, bf8[0:1]
    uef = None
    for l in range(L):
        g = _sc_gather(src, dst, ps_t, pd_t)
        uef = _mm_fuse(x, w, b2, g)
        aggp = _sc_scatter(uef, dst)
        if l < L - 1:
            x, w, b2 = uef, We[l + 1, :D, :], be[l + 1].reshape(1, D)
        a0 = aggp[:N]
        a1 = aggp[NP:NP + N]
        if l < L - 1:
            unf, ps_t, pd_t = _node_update(
                unf, a0, a1, Wn[l, :D, :], Wn[l, D:, :], bn[l].reshape(1, D),
                We[l + 1, D:2 * D, :], We[l + 1, 2 * D:, :])
        else:
            unf = _node_update_last(
                unf, a0, a1, Wn[l, :D, :], Wn[l, D:, :], bn[l].reshape(1, D))
    return unf, uef


# speedup vs baseline: 1.8173x; 1.7428x over previous
import functools

import numpy as np

import jax
import jax.numpy as jnp
from jax import lax
from jax.experimental import pallas as pl
from jax.experimental.pallas import tpu as pltpu
from jax.experimental.pallas import tpu_sc as plsc

N = 10000
E = 320000
ND = 128
ED = 16
D = 128
L = 3

NC = 2
NS = 16
NW = NC * NS
EPW = E // NW
B = 40
ITERS = EPW // B
NP = 10240
RPT = NP // NS

_f32 = jnp.float32


def _mm_bias_body(x_ref, w_ref, b_ref, o_ref):
    o_ref[...] = (
        jnp.dot(x_ref[...], w_ref[...], preferred_element_type=_f32) + b_ref[...]
    )


def _mm_bias(x, w, b2, block_rows):
    m, k = x.shape
    n = w.shape[1]
    return pl.pallas_call(
        _mm_bias_body,
        grid=(m // block_rows,),
        in_specs=[
            pl.BlockSpec((block_rows, k), lambda i: (i, 0)),
            pl.BlockSpec((k, n), lambda i: (0, 0)),
            pl.BlockSpec((1, n), lambda i: (0, 0)),
        ],
        out_specs=pl.BlockSpec((block_rows, n), lambda i: (i, 0)),
        out_shape=jax.ShapeDtypeStruct((m, n), _f32),
    )(x, w, b2)


def _enc_body(nf_ref, w_ref, b_ref, ws_ref, wd_ref, unf_ref, ps_ref, pd_ref):
    u = jnp.dot(nf_ref[...], w_ref[...], preferred_element_type=_f32) + b_ref[...]
    unf_ref[...] = u
    ps_ref[...] = jnp.dot(u, ws_ref[...], preferred_element_type=_f32)
    pd_ref[...] = jnp.dot(u, wd_ref[...], preferred_element_type=_f32)


def _encode_nodes(nf, w, b2, ws, wd, block_rows=2000):
    n_out = jax.ShapeDtypeStruct((N, D), _f32)
    full = pl.BlockSpec((D, D), lambda i: (0, 0))
    return pl.pallas_call(
        _enc_body,
        grid=(N // block_rows,),
        in_specs=[
            pl.BlockSpec((block_rows, ND), lambda i: (i, 0)),
            pl.BlockSpec((ND, D), lambda i: (0, 0)),
            pl.BlockSpec((1, D), lambda i: (0, 0)),
            full, full,
        ],
        out_specs=[pl.BlockSpec((block_rows, D), lambda i: (i, 0))] * 3,
        out_shape=[n_out, n_out, n_out],
    )(nf, w, b2, ws, wd)


def _fold_body(wenc_ref, wee_ref, benc_ref, be_ref, wf_ref, bf_ref):
    wf_ref[...] = jnp.dot(wenc_ref[...], wee_ref[...], preferred_element_type=_f32)
    bf_ref[...] = (
        jnp.dot(benc_ref[...], wee_ref[...], preferred_element_type=_f32) + be_ref[...]
    )


def _fold_edge_encoder(w_edge_enc, we_e0, benc8, be0):
    return pl.pallas_call(
        _fold_body,
        grid=(1,),
        in_specs=[
            pl.BlockSpec((ED, D), lambda i: (0, 0)),
            pl.BlockSpec((D, D), lambda i: (0, 0)),
            pl.BlockSpec((8, D), lambda i: (0, 0)),
            pl.BlockSpec((1, D), lambda i: (0, 0)),
        ],
        out_specs=[
            pl.BlockSpec((ED, D), lambda i: (0, 0)),
            pl.BlockSpec((8, D), lambda i: (0, 0)),
        ],
        out_shape=[
            jax.ShapeDtypeStruct((ED, D), _f32),
            jax.ShapeDtypeStruct((8, D), _f32),
        ],
    )(w_edge_enc, we_e0, benc8, be0)


def _node_body(unf_ref, a0_ref, a1_ref, wn_ref, wa_ref, b_ref, ws_ref, wd_ref,
               out_ref, ps_ref, pd_ref):
    agg = a0_ref[...] + a1_ref[...]
    u = (
        jnp.dot(unf_ref[...], wn_ref[...], preferred_element_type=_f32)
        + jnp.dot(agg, wa_ref[...], preferred_element_type=_f32)
        + b_ref[...]
    )
    u = jnp.maximum(u, 0.0)
    out_ref[...] = u
    ps_ref[...] = jnp.dot(u, ws_ref[...], preferred_element_type=_f32)
    pd_ref[...] = jnp.dot(u, wd_ref[...], preferred_element_type=_f32)


def _node_last_body(unf_ref, a0_ref, a1_ref, wn_ref, wa_ref, b_ref, out_ref):
    agg = a0_ref[...] + a1_ref[...]
    u = (
        jnp.dot(unf_ref[...], wn_ref[...], preferred_element_type=_f32)
        + jnp.dot(agg, wa_ref[...], preferred_element_type=_f32)
        + b_ref[...]
    )
    out_ref[...] = jnp.maximum(u, 0.0)


def _node_update(unf, a0, a1, wn, wa, b2, ws, wd, block_rows=2000):
    n_out = jax.ShapeDtypeStruct((N, D), _f32)
    row = pl.BlockSpec((block_rows, D), lambda i: (i, 0))
    full = pl.BlockSpec((D, D), lambda i: (0, 0))
    return pl.pallas_call(
        _node_body,
        grid=(N // block_rows,),
        in_specs=[row, row, row, full, pl.BlockSpec((D, D), lambda i: (0, 0)),
                  pl.BlockSpec((1, D), lambda i: (0, 0)), full, full],
        out_specs=[row, row, row],
        out_shape=[n_out, n_out, n_out],
    )(unf, a0, a1, wn, wa, b2, ws, wd)


def _node_update_last(unf, a0, a1, wn, wa, b2, block_rows=2000):
    row = pl.BlockSpec((block_rows, D), lambda i: (i, 0))
    full = pl.BlockSpec((D, D), lambda i: (0, 0))
    return pl.pallas_call(
        _node_last_body,
        grid=(N // block_rows,),
        in_specs=[row, row, row, full, pl.BlockSpec((D, D), lambda i: (0, 0)),
                  pl.BlockSpec((1, D), lambda i: (0, 0))],
        out_specs=row,
        out_shape=jax.ShapeDtypeStruct((N, D), _f32),
    )(unf, a0, a1, wn, wa, b2)


def _sc_edge_body(z_hbm, src_hbm, dst_hbm, ps_hbm, pd_hbm,
                  uef_hbm, aggp_hbm,
                  si0_v, si1_v, di0_v, di1_v,
                  z0_v, z1_v, ps0_v, ps1_v, pd0_v, pd1_v,
                  agg_sh,
                  sem_i0, sem_i1, sem_z0, sem_z1, sem_ps0, sem_ps1,
                  sem_pd0, sem_pd1, sem_w0, sem_w1):
    c = lax.axis_index("c")
    s = lax.axis_index("s")
    wid = s * NC + c

    slots = (
        (si0_v, di0_v, z0_v, ps0_v, pd0_v, sem_i0, sem_z0, sem_ps0, sem_pd0,
         sem_w0),
        (si1_v, di1_v, z1_v, ps1_v, pd1_v, sem_i1, sem_z1, sem_ps1, sem_pd1,
         sem_w1),
    )

    def _zrow(r, carry):
        for cc in range(D // 16):
            z0_v[r, pl.ds(cc * 16, 16)] = jnp.zeros((16,), _f32)
        return carry

    lax.fori_loop(0, B, _zrow, 0)
    for j in range(RPT // B):
        pltpu.sync_copy(z0_v, agg_sh.at[pl.ds(s * RPT + j * B, B)])
    plsc.subcore_barrier()

    def _issue_idx(k, slot):
        si_b, di_b = slots[slot][0], slots[slot][1]
        s_i = slots[slot][5]

        @pl.when(k < ITERS)
        def _():
            base = wid * EPW + k * B
            pltpu.async_copy(src_hbm.at[pl.ds(base, B)], si_b, s_i)
            pltpu.async_copy(dst_hbm.at[pl.ds(base, B)], di_b, s_i)

    def _issue_data(k, slot):
        si_b, di_b, z_b, ps_b, pd_b, s_i, s_z, s_ps, s_pd, s_w = slots[slot]

        @pl.when(k < ITERS)
        def _():
            base = wid * EPW + k * B
            pltpu.make_async_copy(src_hbm.at[pl.ds(base, B)], si_b, s_i).wait()
            pltpu.make_async_copy(dst_hbm.at[pl.ds(base, B)], di_b, s_i).wait()

            @pl.when(k >= 2)
            def _():
                pltpu.make_async_copy(z_b, uef_hbm.at[pl.ds(0, B)], s_w).wait()

            pltpu.async_copy(z_hbm.at[pl.ds(base, B)], z_b, s_z)
            pltpu.async_copy(ps_hbm.at[si_b], ps_b, s_ps)
            pltpu.async_copy(pd_hbm.at[di_b], pd_b, s_pd)

    def _stage(k, slot, other):
        si_b, di_b, z_b, ps_b, pd_b, s_i, s_z, s_ps, s_pd, s_w = slots[slot]
        _issue_data(k + 1, other)

        base = wid * EPW + k * B
        pltpu.make_async_copy(z_hbm.at[pl.ds(base, B)], z_b, s_z).wait()
        pltpu.make_async_copy(ps_hbm.at[si_b], ps_b, s_ps).wait()
        pltpu.make_async_copy(pd_hbm.at[di_b], pd_b, s_pd).wait()

        def _row(r, inner):
            for cc in range(D // 16):
                sl = pl.ds(cc * 16, 16)
                v = z_b[r, sl] + ps_b[r, sl] + pd_b[r, sl]
                z_b[r, sl] = jnp.maximum(v, 0.0)
            return inner

        lax.fori_loop(0, B, _row, 0)
        pltpu.async_copy(z_b, uef_hbm.at[pl.ds(base, B)], s_w)
        pltpu.sync_copy(z_b, agg_sh.at[di_b], add=True)
        _issue_idx(k + 2, slot)

    _issue_idx(0, 0)
    _issue_idx(1, 1)
    _issue_data(0, 0)

    def _pair(j, carry):
        k0 = 2 * j
        _stage(k0, 0, 1)
        _stage(k0 + 1, 1, 0)
        return carry

    lax.fori_loop(0, ITERS // 2, _pair, 0)

    pltpu.make_async_copy(z0_v, uef_hbm.at[pl.ds(0, B)], sem_w0).wait()
    pltpu.make_async_copy(z1_v, uef_hbm.at[pl.ds(0, B)], sem_w1).wait()

    plsc.subcore_barrier()
    pltpu.sync_copy(agg_sh.at[pl.ds(s * RPT, RPT)],
                    aggp_hbm.at[pl.ds(c * NP + s * RPT, RPT)])


def _sc_edge(z, src, dst, ps_t, pd_t):
    mesh = plsc.VectorSubcoreMesh(
        core_axis_name="c", subcore_axis_name="s", num_cores=NC, num_subcores=NS
    )
    kern = pl.kernel(
        _sc_edge_body,
        out_type=(
            jax.ShapeDtypeStruct((E, D), _f32),
            jax.ShapeDtypeStruct((NC * NP, D), _f32),
        ),
        mesh=mesh,
        scratch_types=(
            [pltpu.VMEM((B,), jnp.int32)] * 4
            + [pltpu.VMEM((B, D), _f32)] * 6
            + [pltpu.VMEM_SHARED((NP, D), _f32)]
            + [pltpu.SemaphoreType.DMA] * 10
        ),
    )
    return kern(z, src, dst, ps_t, pd_t)


def kernel(nf, ef, edge_index, W_node_enc, b_node_enc, W_edge_enc, b_edge_enc,
           We, be, Wn, bn):
    src = edge_index[0]
    dst = edge_index[1]

    unf, ps_t, pd_t = _encode_nodes(
        nf, W_node_enc, b_node_enc.reshape(1, D),
        We[0, D:2 * D, :], We[0, 2 * D:, :])

    benc8 = jnp.zeros((8, D), _f32).at[0].set(b_edge_enc)
    wf, bf8 = _fold_edge_encoder(W_edge_enc, We[0, :D, :], benc8,
                                 be[0].reshape(1, D))
    z = _mm_bias(ef, wf, bf8[0:1], block_rows=4000)

    uef = None
    for l in range(L):
        if l > 0:
            z = _mm_bias(uef, We[l, :D, :], be[l].reshape(1, D), block_rows=4000)
        uef, aggp = _sc_edge(z, src, dst, ps_t, pd_t)
        a0 = aggp[:N]
        a1 = aggp[NP:NP + N]
        if l < L - 1:
            unf, ps_t, pd_t = _node_update(
                unf, a0, a1, Wn[l, :D, :], Wn[l, D:, :], bn[l].reshape(1, D),
                We[l + 1, D:2 * D, :], We[l + 1, 2 * D:, :])
        else:
            unf = _node_update_last(
                unf, a0, a1, Wn[l, :D, :], Wn[l, D:, :], bn[l].reshape(1, D))
    return unf, uef


# speedup vs baseline: 2.1458x; 1.1808x over previous
import functools

import numpy as np

import jax
import jax.numpy as jnp
from jax import lax
from jax.experimental import pallas as pl
from jax.experimental.pallas import tpu as pltpu
from jax.experimental.pallas import tpu_sc as plsc

N = 10000
E = 320000
ND = 128
ED = 16
D = 128
L = 3

NC = 2
NS = 16
NW = NC * NS
EPW = E // NW
B = 40
ITERS = EPW // B
NP = 10240
RPT = NP // NS

_f32 = jnp.float32


def _mm_bias_body(x_ref, w_ref, b_ref, o_ref):
    o_ref[...] = (
        jnp.dot(x_ref[...], w_ref[...], preferred_element_type=_f32) + b_ref[...]
    )


def _mm_bias(x, w, b2, block_rows):
    m, k = x.shape
    n = w.shape[1]
    return pl.pallas_call(
        _mm_bias_body,
        grid=(m // block_rows,),
        in_specs=[
            pl.BlockSpec((block_rows, k), lambda i: (i, 0)),
            pl.BlockSpec((k, n), lambda i: (0, 0)),
            pl.BlockSpec((1, n), lambda i: (0, 0)),
        ],
        out_specs=pl.BlockSpec((block_rows, n), lambda i: (i, 0)),
        out_shape=jax.ShapeDtypeStruct((m, n), _f32),
    )(x, w, b2)


def _enc_body(nf_ref, w_ref, b_ref, ws_ref, wd_ref, unf_ref, ps_ref, pd_ref):
    u = jnp.dot(nf_ref[...], w_ref[...], preferred_element_type=_f32) + b_ref[...]
    unf_ref[...] = u
    ps_ref[...] = jnp.dot(u, ws_ref[...], preferred_element_type=_f32)
    pd_ref[...] = jnp.dot(u, wd_ref[...], preferred_element_type=_f32)


def _encode_nodes(nf, w, b2, ws, wd, block_rows=2000):
    n_out = jax.ShapeDtypeStruct((N, D), _f32)
    full = pl.BlockSpec((D, D), lambda i: (0, 0))
    return pl.pallas_call(
        _enc_body,
        grid=(N // block_rows,),
        in_specs=[
            pl.BlockSpec((block_rows, ND), lambda i: (i, 0)),
            pl.BlockSpec((ND, D), lambda i: (0, 0)),
            pl.BlockSpec((1, D), lambda i: (0, 0)),
            full, full,
        ],
        out_specs=[pl.BlockSpec((block_rows, D), lambda i: (i, 0))] * 3,
        out_shape=[n_out, n_out, n_out],
    )(nf, w, b2, ws, wd)


def _fold_body(wenc_ref, wee_ref, benc_ref, be_ref, wf_ref, bf_ref):
    wf_ref[...] = jnp.dot(wenc_ref[...], wee_ref[...], preferred_element_type=_f32)
    bf_ref[...] = (
        jnp.dot(benc_ref[...], wee_ref[...], preferred_element_type=_f32) + be_ref[...]
    )


def _fold_edge_encoder(w_edge_enc, we_e0, benc8, be0):
    return pl.pallas_call(
        _fold_body,
        grid=(1,),
        in_specs=[
            pl.BlockSpec((ED, D), lambda i: (0, 0)),
            pl.BlockSpec((D, D), lambda i: (0, 0)),
            pl.BlockSpec((8, D), lambda i: (0, 0)),
            pl.BlockSpec((1, D), lambda i: (0, 0)),
        ],
        out_specs=[
            pl.BlockSpec((ED, D), lambda i: (0, 0)),
            pl.BlockSpec((8, D), lambda i: (0, 0)),
        ],
        out_shape=[
            jax.ShapeDtypeStruct((ED, D), _f32),
            jax.ShapeDtypeStruct((8, D), _f32),
        ],
    )(w_edge_enc, we_e0, benc8, be0)


def _node_body(unf_ref, a0_ref, a1_ref, wn_ref, wa_ref, b_ref, ws_ref, wd_ref,
               out_ref, ps_ref, pd_ref):
    agg = a0_ref[...] + a1_ref[...]
    u = (
        jnp.dot(unf_ref[...], wn_ref[...], preferred_element_type=_f32)
        + jnp.dot(agg, wa_ref[...], preferred_element_type=_f32)
        + b_ref[...]
    )
    u = jnp.maximum(u, 0.0)
    out_ref[...] = u
    ps_ref[...] = jnp.dot(u, ws_ref[...], preferred_element_type=_f32)
    pd_ref[...] = jnp.dot(u, wd_ref[...], preferred_element_type=_f32)


def _node_last_body(unf_ref, a0_ref, a1_ref, wn_ref, wa_ref, b_ref, out_ref):
    agg = a0_ref[...] + a1_ref[...]
    u = (
        jnp.dot(unf_ref[...], wn_ref[...], preferred_element_type=_f32)
        + jnp.dot(agg, wa_ref[...], preferred_element_type=_f32)
        + b_ref[...]
    )
    out_ref[...] = jnp.maximum(u, 0.0)


def _node_update(unf, a0, a1, wn, wa, b2, ws, wd, block_rows=2000):
    n_out = jax.ShapeDtypeStruct((N, D), _f32)
    row = pl.BlockSpec((block_rows, D), lambda i: (i, 0))
    full = pl.BlockSpec((D, D), lambda i: (0, 0))
    return pl.pallas_call(
        _node_body,
        grid=(N // block_rows,),
        in_specs=[row, row, row, full, pl.BlockSpec((D, D), lambda i: (0, 0)),
                  pl.BlockSpec((1, D), lambda i: (0, 0)), full, full],
        out_specs=[row, row, row],
        out_shape=[n_out, n_out, n_out],
    )(unf, a0, a1, wn, wa, b2, ws, wd)


def _node_update_last(unf, a0, a1, wn, wa, b2, block_rows=2000):
    row = pl.BlockSpec((block_rows, D), lambda i: (i, 0))
    full = pl.BlockSpec((D, D), lambda i: (0, 0))
    return pl.pallas_call(
        _node_last_body,
        grid=(N // block_rows,),
        in_specs=[row, row, row, full, pl.BlockSpec((D, D), lambda i: (0, 0)),
                  pl.BlockSpec((1, D), lambda i: (0, 0))],
        out_specs=row,
        out_shape=jax.ShapeDtypeStruct((N, D), _f32),
    )(unf, a0, a1, wn, wa, b2)


def _sc_edge_body(z_hbm, src_hbm, dst_hbm, ps_hbm, pd_hbm,
                  uef_hbm, aggp_hbm,
                  si0_v, si1_v, si2_v, si3_v, di0_v, di1_v, di2_v, di3_v,
                  z0_v, z1_v, ps0_v, ps1_v, pd0_v, pd1_v, o0_v, o1_v,
                  agg_sh,
                  sem_i0, sem_i1, sem_i2, sem_i3,
                  sem_z0, sem_z1, sem_ps0, sem_ps1, sem_pd0, sem_pd1,
                  sem_w0, sem_w1, sem_s0, sem_s1):
    c = lax.axis_index("c")
    s = lax.axis_index("s")
    wid = s * NC + c

    si = (si0_v, si1_v, si2_v, si3_v)
    di = (di0_v, di1_v, di2_v, di3_v)
    sem_i = (sem_i0, sem_i1, sem_i2, sem_i3)
    zb = (z0_v, z1_v)
    psb = (ps0_v, ps1_v)
    pdb = (pd0_v, pd1_v)
    ob = (o0_v, o1_v)
    sem_z = (sem_z0, sem_z1)
    sem_ps = (sem_ps0, sem_ps1)
    sem_pd = (sem_pd0, sem_pd1)
    sem_w = (sem_w0, sem_w1)
    sem_s = (sem_s0, sem_s1)

    def _zrow(r, carry):
        for cc in range(D // 16):
            o0_v[r, pl.ds(cc * 16, 16)] = jnp.zeros((16,), _f32)
        return carry

    lax.fori_loop(0, B, _zrow, 0)
    for j in range(RPT // B):
        pltpu.sync_copy(o0_v, agg_sh.at[pl.ds(s * RPT + j * B, B)])
    plsc.subcore_barrier()

    def _issue_idx(k, m):
        @pl.when(k < ITERS)
        def _():
            base = wid * EPW + k * B
            pltpu.async_copy(src_hbm.at[pl.ds(base, B)], si[m], sem_i[m])
            pltpu.async_copy(dst_hbm.at[pl.ds(base, B)], di[m], sem_i[m])

    def _issue_data(k, d, m):
        @pl.when(k < ITERS)
        def _():
            base = wid * EPW + k * B
            pltpu.make_async_copy(src_hbm.at[pl.ds(base, B)], si[m], sem_i[m]).wait()
            pltpu.make_async_copy(dst_hbm.at[pl.ds(base, B)], di[m], sem_i[m]).wait()
            pltpu.async_copy(z_hbm.at[pl.ds(base, B)], zb[d], sem_z[d])
            pltpu.async_copy(ps_hbm.at[si[m]], psb[d], sem_ps[d])
            pltpu.async_copy(pd_hbm.at[di[m]], pdb[d], sem_pd[d])

    def _stage(k, d, m):
        _issue_data(k + 1, 1 - d, (m + 1) % 4)

        base = wid * EPW + k * B
        pltpu.make_async_copy(z_hbm.at[pl.ds(base, B)], zb[d], sem_z[d]).wait()
        pltpu.make_async_copy(ps_hbm.at[si[m]], psb[d], sem_ps[d]).wait()
        pltpu.make_async_copy(pd_hbm.at[di[m]], pdb[d], sem_pd[d]).wait()

        @pl.when(k >= 2)
        def _():
            pltpu.make_async_copy(ob[d], uef_hbm.at[pl.ds(0, B)], sem_w[d]).wait()
            pltpu.make_async_copy(ob[d], agg_sh.at[di[(m + 2) % 4]], sem_s[d]).wait()

        _issue_idx(k + 2, (m + 2) % 4)

        def _row(r, inner):
            for cc in range(D // 16):
                sl = pl.ds(cc * 16, 16)
                v = zb[d][r, sl] + psb[d][r, sl] + pdb[d][r, sl]
                ob[d][r, sl] = jnp.maximum(v, 0.0)
            return inner

        lax.fori_loop(0, B, _row, 0)
        pltpu.async_copy(ob[d], uef_hbm.at[pl.ds(base, B)], sem_w[d])
        pltpu.async_copy(ob[d], agg_sh.at[di[m]], sem_s[d], add=True)

    _issue_idx(0, 0)
    _issue_idx(1, 1)
    _issue_data(0, 0, 0)

    def _quad(j, carry):
        k0 = 4 * j
        _stage(k0, 0, 0)
        _stage(k0 + 1, 1, 1)
        _stage(k0 + 2, 0, 2)
        _stage(k0 + 3, 1, 3)
        return carry

    lax.fori_loop(0, ITERS // 4, _quad, 0)
    _stage(ITERS - 2, 0, 0)
    _stage(ITERS - 1, 1, 1)

    pltpu.make_async_copy(o0_v, uef_hbm.at[pl.ds(0, B)], sem_w0).wait()
    pltpu.make_async_copy(o0_v, agg_sh.at[di0_v], sem_s0).wait()
    pltpu.make_async_copy(o1_v, uef_hbm.at[pl.ds(0, B)], sem_w1).wait()
    pltpu.make_async_copy(o1_v, agg_sh.at[di1_v], sem_s1).wait()

    plsc.subcore_barrier()
    pltpu.sync_copy(agg_sh.at[pl.ds(s * RPT, RPT)],
                    aggp_hbm.at[pl.ds(c * NP + s * RPT, RPT)])


def _sc_edge(z, src, dst, ps_t, pd_t):
    mesh = plsc.VectorSubcoreMesh(
        core_axis_name="c", subcore_axis_name="s", num_cores=NC, num_subcores=NS
    )
    kern = pl.kernel(
        _sc_edge_body,
        out_type=(
            jax.ShapeDtypeStruct((E, D), _f32),
            jax.ShapeDtypeStruct((NC * NP, D), _f32),
        ),
        mesh=mesh,
        scratch_types=(
            [pltpu.VMEM((B,), jnp.int32)] * 8
            + [pltpu.VMEM((B, D), _f32)] * 8
            + [pltpu.VMEM_SHARED((NP, D), _f32)]
            + [pltpu.SemaphoreType.DMA] * 14
        ),
    )
    return kern(z, src, dst, ps_t, pd_t)


def kernel(nf, ef, edge_index, W_node_enc, b_node_enc, W_edge_enc, b_edge_enc,
           We, be, Wn, bn):
    src = edge_index[0]
    dst = edge_index[1]

    unf, ps_t, pd_t = _encode_nodes(
        nf, W_node_enc, b_node_enc.reshape(1, D),
        We[0, D:2 * D, :], We[0, 2 * D:, :])

    benc8 = jnp.zeros((8, D), _f32).at[0].set(b_edge_enc)
    wf, bf8 = _fold_edge_encoder(W_edge_enc, We[0, :D, :], benc8,
                                 be[0].reshape(1, D))
    z = _mm_bias(ef, wf, bf8[0:1], block_rows=4000)

    uef = None
    for l in range(L):
        if l > 0:
            z = _mm_bias(uef, We[l, :D, :], be[l].reshape(1, D), block_rows=4000)
        uef, aggp = _sc_edge(z, src, dst, ps_t, pd_t)
        a0 = aggp[:N]
        a1 = aggp[NP:NP + N]
        if l < L - 1:
            unf, ps_t, pd_t = _node_update(
                unf, a0, a1, Wn[l, :D, :], Wn[l, D:, :], bn[l].reshape(1, D),
                We[l + 1, D:2 * D, :], We[l + 1, 2 * D:, :])
        else:
            unf = _node_update_last(
                unf, a0, a1, Wn[l, :D, :], Wn[l, D:, :], bn[l].reshape(1, D))
    return unf, uef


# speedup vs baseline: 2.2287x; 1.0386x over previous
import functools

import numpy as np

import jax
import jax.numpy as jnp
from jax import lax
from jax.experimental import pallas as pl
from jax.experimental.pallas import tpu as pltpu
from jax.experimental.pallas import tpu_sc as plsc

N = 10000
E = 320000
ND = 128
ED = 16
D = 128
L = 3

NC = 2
NS = 16
NW = NC * NS
EPW = E // NW
B = 40
ITERS = EPW // B
NP = 10240
RPT = NP // NS

_f32 = jnp.float32


def _mm_bias_body(x_ref, w_ref, b_ref, o_ref):
    o_ref[...] = (
        jnp.dot(x_ref[...], w_ref[...], preferred_element_type=_f32) + b_ref[...]
    )


def _mm_bias(x, w, b2, block_rows):
    m, k = x.shape
    n = w.shape[1]
    return pl.pallas_call(
        _mm_bias_body,
        grid=(m // block_rows,),
        in_specs=[
            pl.BlockSpec((block_rows, k), lambda i: (i, 0)),
            pl.BlockSpec((k, n), lambda i: (0, 0)),
            pl.BlockSpec((1, n), lambda i: (0, 0)),
        ],
        out_specs=pl.BlockSpec((block_rows, n), lambda i: (i, 0)),
        out_shape=jax.ShapeDtypeStruct((m, n), _f32),
    )(x, w, b2)


def _enc_body(nf_ref, w_ref, b_ref, ws_ref, wd_ref, unf_ref, ps_ref, pd_ref):
    u = jnp.dot(nf_ref[...], w_ref[...], preferred_element_type=_f32) + b_ref[...]
    unf_ref[...] = u
    ps_ref[...] = jnp.dot(u, ws_ref[...], preferred_element_type=_f32)
    pd_ref[...] = jnp.dot(u, wd_ref[...], preferred_element_type=_f32)


def _encode_nodes(nf, w, b2, ws, wd, block_rows=5000):
    n_out = jax.ShapeDtypeStruct((N, D), _f32)
    full = pl.BlockSpec((D, D), lambda i: (0, 0))
    return pl.pallas_call(
        _enc_body,
        grid=(N // block_rows,),
        in_specs=[
            pl.BlockSpec((block_rows, ND), lambda i: (i, 0)),
            pl.BlockSpec((ND, D), lambda i: (0, 0)),
            pl.BlockSpec((1, D), lambda i: (0, 0)),
            full, full,
        ],
        out_specs=[pl.BlockSpec((block_rows, D), lambda i: (i, 0))] * 3,
        out_shape=[n_out, n_out, n_out],
    )(nf, w, b2, ws, wd)


def _fold_body(wenc_ref, wee_ref, benc_ref, be_ref, wf_ref, bf_ref):
    wf_ref[...] = jnp.dot(wenc_ref[...], wee_ref[...], preferred_element_type=_f32)
    bf_ref[...] = (
        jnp.dot(benc_ref[...], wee_ref[...], preferred_element_type=_f32) + be_ref[...]
    )


def _fold_edge_encoder(w_edge_enc, we_e0, benc8, be0):
    return pl.pallas_call(
        _fold_body,
        grid=(1,),
        in_specs=[
            pl.BlockSpec((ED, D), lambda i: (0, 0)),
            pl.BlockSpec((D, D), lambda i: (0, 0)),
            pl.BlockSpec((8, D), lambda i: (0, 0)),
            pl.BlockSpec((1, D), lambda i: (0, 0)),
        ],
        out_specs=[
            pl.BlockSpec((ED, D), lambda i: (0, 0)),
            pl.BlockSpec((8, D), lambda i: (0, 0)),
        ],
        out_shape=[
            jax.ShapeDtypeStruct((ED, D), _f32),
            jax.ShapeDtypeStruct((8, D), _f32),
        ],
    )(w_edge_enc, we_e0, benc8, be0)


def _node_body(unf_ref, a0_ref, a1_ref, wn_ref, wa_ref, b_ref, ws_ref, wd_ref,
               out_ref, ps_ref, pd_ref):
    agg = a0_ref[...] + a1_ref[...]
    u = (
        jnp.dot(unf_ref[...], wn_ref[...], preferred_element_type=_f32)
        + jnp.dot(agg, wa_ref[...], preferred_element_type=_f32)
        + b_ref[...]
    )
    u = jnp.maximum(u, 0.0)
    out_ref[...] = u
    ps_ref[...] = jnp.dot(u, ws_ref[...], preferred_element_type=_f32)
    pd_ref[...] = jnp.dot(u, wd_ref[...], preferred_element_type=_f32)


def _node_last_body(unf_ref, a0_ref, a1_ref, wn_ref, wa_ref, b_ref, out_ref):
    agg = a0_ref[...] + a1_ref[...]
    u = (
        jnp.dot(unf_ref[...], wn_ref[...], preferred_element_type=_f32)
        + jnp.dot(agg, wa_ref[...], preferred_element_type=_f32)
        + b_ref[...]
    )
    out_ref[...] = jnp.maximum(u, 0.0)


def _node_update(unf, a0, a1, wn, wa, b2, ws, wd, block_rows=5000):
    n_out = jax.ShapeDtypeStruct((N, D), _f32)
    row = pl.BlockSpec((block_rows, D), lambda i: (i, 0))
    full = pl.BlockSpec((D, D), lambda i: (0, 0))
    return pl.pallas_call(
        _node_body,
        grid=(N // block_rows,),
        in_specs=[row, row, row, full, pl.BlockSpec((D, D), lambda i: (0, 0)),
                  pl.BlockSpec((1, D), lambda i: (0, 0)), full, full],
        out_specs=[row, row, row],
        out_shape=[n_out, n_out, n_out],
    )(unf, a0, a1, wn, wa, b2, ws, wd)


def _node_update_last(unf, a0, a1, wn, wa, b2, block_rows=5000):
    row = pl.BlockSpec((block_rows, D), lambda i: (i, 0))
    full = pl.BlockSpec((D, D), lambda i: (0, 0))
    return pl.pallas_call(
        _node_last_body,
        grid=(N // block_rows,),
        in_specs=[row, row, row, full, pl.BlockSpec((D, D), lambda i: (0, 0)),
                  pl.BlockSpec((1, D), lambda i: (0, 0))],
        out_specs=row,
        out_shape=jax.ShapeDtypeStruct((N, D), _f32),
    )(unf, a0, a1, wn, wa, b2)


def _sc_edge_body(z_hbm, src_hbm, dst_hbm, ps_hbm, pd_hbm,
                  uef_hbm, aggp_hbm,
                  si0_v, si1_v, si2_v, si3_v, di0_v, di1_v, di2_v, di3_v,
                  z0_v, z1_v, ps0_v, ps1_v, pd0_v, pd1_v, o0_v, o1_v,
                  agg_sh,
                  sem_i0, sem_i1, sem_i2, sem_i3,
                  sem_z0, sem_z1, sem_ps0, sem_ps1, sem_pd0, sem_pd1,
                  sem_w0, sem_w1, sem_s0, sem_s1):
    c = lax.axis_index("c")
    s = lax.axis_index("s")
    wid = s * NC + c

    si = (si0_v, si1_v, si2_v, si3_v)
    di = (di0_v, di1_v, di2_v, di3_v)
    sem_i = (sem_i0, sem_i1, sem_i2, sem_i3)
    zb = (z0_v, z1_v)
    psb = (ps0_v, ps1_v)
    pdb = (pd0_v, pd1_v)
    ob = (o0_v, o1_v)
    sem_z = (sem_z0, sem_z1)
    sem_ps = (sem_ps0, sem_ps1)
    sem_pd = (sem_pd0, sem_pd1)
    sem_w = (sem_w0, sem_w1)
    sem_s = (sem_s0, sem_s1)

    def _zrow(r, carry):
        for cc in range(D // 16):
            o0_v[r, pl.ds(cc * 16, 16)] = jnp.zeros((16,), _f32)
        return carry

    lax.fori_loop(0, B, _zrow, 0)
    for j in range(RPT // B):
        pltpu.sync_copy(o0_v, agg_sh.at[pl.ds(s * RPT + j * B, B)])
    plsc.subcore_barrier()

    def _issue_idx(k, m):
        @pl.when(k < ITERS)
        def _():
            base = wid * EPW + k * B
            pltpu.async_copy(src_hbm.at[pl.ds(base, B)], si[m], sem_i[m])
            pltpu.async_copy(dst_hbm.at[pl.ds(base, B)], di[m], sem_i[m])

    def _issue_data(k, d, m):
        @pl.when(k < ITERS)
        def _():
            base = wid * EPW + k * B
            pltpu.make_async_copy(src_hbm.at[pl.ds(base, B)], si[m], sem_i[m]).wait()
            pltpu.make_async_copy(dst_hbm.at[pl.ds(base, B)], di[m], sem_i[m]).wait()
            pltpu.async_copy(z_hbm.at[pl.ds(base, B)], zb[d], sem_z[d])
            pltpu.async_copy(ps_hbm.at[si[m]], psb[d], sem_ps[d])
            pltpu.async_copy(pd_hbm.at[di[m]], pdb[d], sem_pd[d])

    def _stage(k, d, m):
        _issue_data(k + 1, 1 - d, (m + 1) % 4)

        base = wid * EPW + k * B
        pltpu.make_async_copy(z_hbm.at[pl.ds(base, B)], zb[d], sem_z[d]).wait()
        pltpu.make_async_copy(ps_hbm.at[si[m]], psb[d], sem_ps[d]).wait()
        pltpu.make_async_copy(pd_hbm.at[di[m]], pdb[d], sem_pd[d]).wait()

        @pl.when(k >= 2)
        def _():
            pltpu.make_async_copy(ob[d], uef_hbm.at[pl.ds(0, B)], sem_w[d]).wait()
            pltpu.make_async_copy(ob[d], agg_sh.at[di[(m + 2) % 4]], sem_s[d]).wait()

        _issue_idx(k + 2, (m + 2) % 4)

        def _row(r, inner):
            for cc in range(D // 16):
                sl = pl.ds(cc * 16, 16)
                v = zb[d][r, sl] + psb[d][r, sl] + pdb[d][r, sl]
                ob[d][r, sl] = jnp.maximum(v, 0.0)
            return inner

        lax.fori_loop(0, B, _row, 0)
        pltpu.async_copy(ob[d], uef_hbm.at[pl.ds(base, B)], sem_w[d])
        pltpu.async_copy(ob[d], agg_sh.at[di[m]], sem_s[d], add=True)

    _issue_idx(0, 0)
    _issue_idx(1, 1)
    _issue_data(0, 0, 0)

    def _quad(j, carry):
        k0 = 4 * j
        _stage(k0, 0, 0)
        _stage(k0 + 1, 1, 1)
        _stage(k0 + 2, 0, 2)
        _stage(k0 + 3, 1, 3)
        return carry

    lax.fori_loop(0, ITERS // 4, _quad, 0)
    _stage(ITERS - 2, 0, 0)
    _stage(ITERS - 1, 1, 1)

    pltpu.make_async_copy(o0_v, uef_hbm.at[pl.ds(0, B)], sem_w0).wait()
    pltpu.make_async_copy(o0_v, agg_sh.at[di0_v], sem_s0).wait()
    pltpu.make_async_copy(o1_v, uef_hbm.at[pl.ds(0, B)], sem_w1).wait()
    pltpu.make_async_copy(o1_v, agg_sh.at[di1_v], sem_s1).wait()

    plsc.subcore_barrier()
    pltpu.sync_copy(agg_sh.at[pl.ds(s * RPT, RPT)],
                    aggp_hbm.at[pl.ds(c * NP + s * RPT, RPT)])


def _sc_edge(z, src, dst, ps_t, pd_t):
    mesh = plsc.VectorSubcoreMesh(
        core_axis_name="c", subcore_axis_name="s", num_cores=NC, num_subcores=NS
    )
    kern = pl.kernel(
        _sc_edge_body,
        out_type=(
            jax.ShapeDtypeStruct((E, D), _f32),
            jax.ShapeDtypeStruct((NC * NP, D), _f32),
        ),
        mesh=mesh,
        scratch_types=(
            [pltpu.VMEM((B,), jnp.int32)] * 8
            + [pltpu.VMEM((B, D), _f32)] * 8
            + [pltpu.VMEM_SHARED((NP, D), _f32)]
            + [pltpu.SemaphoreType.DMA] * 14
        ),
    )
    return kern(z, src, dst, ps_t, pd_t)


def kernel(nf, ef, edge_index, W_node_enc, b_node_enc, W_edge_enc, b_edge_enc,
           We, be, Wn, bn):
    src = edge_index[0]
    dst = edge_index[1]

    unf, ps_t, pd_t = _encode_nodes(
        nf, W_node_enc, b_node_enc.reshape(1, D),
        We[0, D:2 * D, :], We[0, 2 * D:, :])

    benc8 = jnp.zeros((8, D), _f32).at[0].set(b_edge_enc)
    wf, bf8 = _fold_edge_encoder(W_edge_enc, We[0, :D, :], benc8,
                                 be[0].reshape(1, D))
    z = _mm_bias(ef, wf, bf8[0:1], block_rows=8000)

    uef = None
    for l in range(L):
        if l > 0:
            z = _mm_bias(uef, We[l, :D, :], be[l].reshape(1, D), block_rows=8000)
        uef, aggp = _sc_edge(z, src, dst, ps_t, pd_t)
        a0 = aggp[:N]
        a1 = aggp[NP:NP + N]
        if l < L - 1:
            unf, ps_t, pd_t = _node_update(
                unf, a0, a1, Wn[l, :D, :], Wn[l, D:, :], bn[l].reshape(1, D),
                We[l + 1, D:2 * D, :], We[l + 1, 2 * D:, :])
        else:
            unf = _node_update_last(
                unf, a0, a1, Wn[l, :D, :], Wn[l, D:, :], bn[l].reshape(1, D))
    return unf, uef
